# Initial kernel scaffold; baseline (speedup 1.0000x reference)
import jax
import jax.numpy as jnp
from jax.experimental import pallas as pl


def kernel(traj_pos, traj_backbone_frame, single_mask, single_res_rel, aatype, params):
    raise NotImplementedError("write your pallas kernel here")


# speedup vs baseline: 13.0572x; 13.0572x over previous
import functools

import jax
import jax.numpy as jnp
import numpy as np
from jax import lax
from jax.experimental import pallas as pl

EMBED = 256
NS, NV = 100, 16
ES, EV = 32, 1
TOPK = 16
EPS = 1e-8
GW = 160
NB = 64
NBN = 256
_INTERPRET = False


def _norm(v, axis=-1, keepdims=False):
    return jnp.sqrt(jnp.sum(v * v, axis=axis, keepdims=keepdims) + EPS)


def _normalize(v, axis=-1):
    return v / _norm(v, axis=axis, keepdims=True)


def _rbf(d, d_min, d_max, n):
    mu = jnp.linspace(d_min, d_max, n)
    sigma = (d_max - d_min) / n
    return jnp.exp(-((d[..., None] - mu) / sigma) ** 2)


def _dihedral_feats(X):
    Bb, Ll = X.shape[0], X.shape[1]
    Xf = X.reshape(Bb, Ll * 3, 3)
    dX = Xf[:, 1:] - Xf[:, :-1]
    U = _normalize(dX)
    u2, u1, u0 = U[:, :-2], U[:, 1:-1], U[:, 2:]
    n2 = _normalize(jnp.cross(u2, u1))
    n1 = _normalize(jnp.cross(u1, u0))
    cosD = jnp.clip(jnp.sum(n2 * n1, -1), -1 + 1e-7, 1 - 1e-7)
    D = jnp.sign(jnp.sum(u2 * n1, -1)) * jnp.arccos(cosD)
    D = jnp.pad(D, ((0, 0), (1, 2)))
    D = D.reshape(Bb, Ll, 3)
    return jnp.concatenate([jnp.cos(D), jnp.sin(D)], -1)


def _orientation_vecs(X):
    ca = X[:, :, 1]
    f = _normalize(ca[:, 1:] - ca[:, :-1])
    b = _normalize(ca[:, :-1] - ca[:, 1:])
    f = jnp.pad(f, ((0, 0), (0, 1), (0, 0)))
    b = jnp.pad(b, ((0, 0), (1, 0), (0, 0)))
    n_ = _normalize(X[:, :, 0] - ca)
    c_ = _normalize(X[:, :, 2] - ca)
    bis = _normalize(c_ + n_)
    perp = _normalize(jnp.cross(c_, n_))
    side = -bis * jnp.sqrt(1.0 / 3) - perp * jnp.sqrt(2.0 / 3)
    return jnp.stack([f, b, side], axis=-2)


def _rotation_frames(X):
    n, ca, c = X[:, :, 0], X[:, :, 1], X[:, :, 2]
    v1 = c - ca
    v2 = n - ca
    e1 = _normalize(v1)
    u2 = v2 - e1 * jnp.sum(e1 * v2, -1, keepdims=True)
    e2 = _normalize(u2)
    e3 = jnp.cross(e1, e2)
    return jnp.stack([e1, e2, e3], axis=-1)


def _k_ln_s(x, n):
    m = jnp.sum(x, axis=1, keepdims=True) / n
    d = x - m
    var = jnp.sum(d * d, axis=1, keepdims=True) / n
    return d * lax.rsqrt(var + 1e-5)


def _k_ln_v(vx, vy, vz, nv):
    n2 = jnp.sum(vx * vx + vy * vy + vz * vz, axis=1, keepdims=True) / nv
    inv = lax.rsqrt(n2 + EPS)
    return vx * inv, vy * inv, vz * inv


def _k_gvp(s, vx, vy, vz, Wh, Ws, bs, Wmu, Wg, bg, act):
    vhx = jnp.dot(vx, Wh, preferred_element_type=jnp.float32)
    vhy = jnp.dot(vy, Wh, preferred_element_type=jnp.float32)
    vhz = jnp.dot(vz, Wh, preferred_element_type=jnp.float32)
    vn = jnp.sqrt(vhx * vhx + vhy * vhy + vhz * vhz + EPS)
    so = jnp.dot(jnp.concatenate([s, vn], axis=1), Ws,
                 preferred_element_type=jnp.float32) + bs
    gate = jax.nn.sigmoid(jnp.dot(so, Wg, preferred_element_type=jnp.float32) + bg)
    vox = jnp.dot(vhx, Wmu, preferred_element_type=jnp.float32) * gate
    voy = jnp.dot(vhy, Wmu, preferred_element_type=jnp.float32) * gate
    voz = jnp.dot(vhz, Wmu, preferred_element_type=jnp.float32) * gate
    if act:
        so = jax.nn.relu(so)
    return so, vox, voy, voz


def _node_body(in_ref, Wh, Ws, bs, Wmu, Wg, bg, out_ref):
    x = in_ref[...]
    s = x[:, 0:22]
    vx, vy, vz = x[:, 22:25], x[:, 25:28], x[:, 28:31]
    carr = x[:, 31:35]
    so, vox, voy, voz = _k_gvp(s, vx, vy, vz, Wh[...], Ws[...], bs[...],
                               Wmu[...], Wg[...], bg[...], act=True)
    so = _k_ln_s(so, NS)
    vox, voy, voz = _k_ln_v(vox, voy, voz, NV)
    pad = jnp.zeros((x.shape[0], GW - NS - 3 * NV - 4), jnp.float32)
    out_ref[...] = jnp.concatenate([so, vox, voy, voz, carr, pad], axis=1)


def _edge_body(car_ref, grow_ref, gblk_ref, misc_ref, *rest):
    (Whe, Wse, bse, Wmue, Wge, bge,
     Wh1, Ws1, bs1, Wmu1, Wg1, bg1,
     Wh2, Ws2, bs2, Wmu2, Wg2, bg2,
     Wh3, Ws3, bs3, Wmu3, Wg3, bg3,
     Whf1, Wsf1, bsf1, Wmuf1, Wgf1, bgf1,
     Whf2, Wsf2, bsf2, Wmuf2, Wgf2, bgf2,
     Wos, Wox, Woy, Woz, Wdc, Wix, Wiy, Wiz, Wemb, bsum,
     out_ref) = rest

    blk = pl.program_id(1)
    Ll = car_ref.shape[2]

    car = car_ref[0]
    grow = grow_ref[0]
    gblk = gblk_ref[0]
    misc = misc_ref[0]

    cax, cay, caz = car[0:1, :], car[1:2, :], car[2:3, :]
    cbx = gblk[:, 148:149]
    cby = gblk[:, 149:150]
    cbz = gblk[:, 150:151]

    dx = cbx - cax
    dy = cby - cay
    dz = cbz - caz
    d2 = (dx * dx + dy * dy) + dz * dz

    ci = lax.broadcasted_iota(jnp.int32, (NB, Ll), 1)
    nid = blk * NB + lax.broadcasted_iota(jnp.int32, (NB, 1), 0)
    d2 = jnp.where(ci == nid, d2 + 1e9, d2)

    big = jnp.int32(2 ** 30)
    gat_list = []
    for k in range(TOPK):
        m = jnp.min(d2, axis=1, keepdims=True)
        eq = d2 == m
        idx = jnp.min(jnp.where(eq, ci, big), axis=1, keepdims=True)
        oh = ci == idx
        gat_list.append(jnp.dot(oh.astype(jnp.float32), grow,
                                preferred_element_type=jnp.float32))
        if k < TOPK - 1:
            d2 = jnp.where(oh, jnp.float32(1e9), d2)
    gat = jnp.concatenate(gat_list, axis=0)

    src_s = gat[:, 0:NS]
    svx, svy, svz = gat[:, 100:116], gat[:, 116:132], gat[:, 132:148]
    scx, scy, scz = gat[:, 148:149], gat[:, 149:150], gat[:, 150:151]
    sres = gat[:, 151:152]

    tile16 = lambda a: jnp.concatenate([a] * TOPK, axis=0)
    s_d = tile16(gblk[:, 0:NS])
    dvx, dvy, dvz = (tile16(gblk[:, 100:116]), tile16(gblk[:, 116:132]),
                     tile16(gblk[:, 132:148]))
    dcx, dcy, dcz = (tile16(cbx), tile16(cby), tile16(cbz))
    dres = tile16(gblk[:, 151:152])

    ex = scx - dcx
    ey = scy - dcy
    ez = scz - dcz
    dist = jnp.sqrt(ex * ex + ey * ey + ez * ez + EPS)
    mu = lax.broadcasted_iota(jnp.int32, (1, 16), 1).astype(jnp.float32) * (20.0 / 15.0)
    sig = 20.0 / 16.0
    e_rbf = jnp.exp(-(((dist - mu) / sig) ** 2))
    doff = sres - dres
    freqs = jnp.exp(lax.broadcasted_iota(jnp.int32, (1, 8), 1).astype(jnp.float32)
                    * (-np.log(10000.0) / 8))
    ang = doff * freqs
    e_s_in = jnp.concatenate([e_rbf, jnp.sin(ang), jnp.cos(ang)], axis=1)
    inv_d = 1.0 / dist
    evx, evy, evz = ex * inv_d, ey * inv_d, ez * inv_d

    vhx, vhy, vhz = evx * Whe[...], evy * Whe[...], evz * Whe[...]
    vn = jnp.sqrt(vhx * vhx + vhy * vhy + vhz * vhz + EPS)
    so = jnp.dot(jnp.concatenate([e_s_in, vn], axis=1), Wse[...],
                 preferred_element_type=jnp.float32) + bse[...]
    gate = jax.nn.sigmoid(jnp.dot(so, Wge[...],
                                  preferred_element_type=jnp.float32) + bge[...])
    eox = vhx * Wmue[...] * gate
    eoy = vhy * Wmue[...] * gate
    eoz = vhz * Wmue[...] * gate
    so = jax.nn.relu(so)
    e_s = _k_ln_s(so, ES)
    eox, eoy, eoz = _k_ln_v(eox, eoy, eoz, EV)

    ms = jnp.concatenate([src_s, e_s, s_d], axis=1)
    mvx = jnp.concatenate([svx, eox, dvx], axis=1)
    mvy = jnp.concatenate([svy, eoy, dvy], axis=1)
    mvz = jnp.concatenate([svz, eoz, dvz], axis=1)
    ms, mvx, mvy, mvz = _k_gvp(ms, mvx, mvy, mvz, Wh1[...], Ws1[...], bs1[...],
                               Wmu1[...], Wg1[...], bg1[...], act=True)
    ms, mvx, mvy, mvz = _k_gvp(ms, mvx, mvy, mvz, Wh2[...], Ws2[...], bs2[...],
                               Wmu2[...], Wg2[...], bg2[...], act=True)
    ms, mvx, mvy, mvz = _k_gvp(ms, mvx, mvy, mvz, Wh3[...], Ws3[...], bs3[...],
                               Wmu3[...], Wg3[...], bg3[...], act=False)

    def agg(a):
        t = a[0:NB]
        for k in range(1, TOPK):
            t = t + a[k * NB:(k + 1) * NB]
        return t * (1.0 / TOPK)

    s1 = _k_ln_s(gblk[:, 0:NS] + agg(ms), NS)
    v1x, v1y, v1z = _k_ln_v(gblk[:, 100:116] + agg(mvx),
                            gblk[:, 116:132] + agg(mvy),
                            gblk[:, 132:148] + agg(mvz), NV)

    fs, fvx, fvy, fvz = _k_gvp(s1, v1x, v1y, v1z, Whf1[...], Wsf1[...],
                               bsf1[...], Wmuf1[...], Wgf1[...], bgf1[...],
                               act=True)
    fs, fvx, fvy, fvz = _k_gvp(fs, fvx, fvy, fvz, Whf2[...], Wsf2[...],
                               bsf2[...], Wmuf2[...], Wgf2[...], bgf2[...],
                               act=False)
    s2 = _k_ln_s(s1 + fs, NS)
    v2x, v2y, v2z = _k_ln_v(v1x + fvx, v1y + fvy, v1z + fvz, NV)

    dih = misc[:, 0:6]
    vfx, vfy, vfz = misc[:, 6:9], misc[:, 9:12], misc[:, 12:15]
    Rt = [[misc[:, 15 + 3 * i + j:16 + 3 * i + j] for j in range(3)]
          for i in range(3)]
    aat = misc[:, 24:46]

    def rot(ax, ay, az, j):
        return ax * Rt[0][j] + ay * Rt[1][j] + az * Rt[2][j]

    res = jnp.dot(s2, Wos[...], preferred_element_type=jnp.float32)
    res = res + jnp.dot(rot(v2x, v2y, v2z, 0), Wox[...],
                        preferred_element_type=jnp.float32)
    res = res + jnp.dot(rot(v2x, v2y, v2z, 1), Woy[...],
                        preferred_element_type=jnp.float32)
    res = res + jnp.dot(rot(v2x, v2y, v2z, 2), Woz[...],
                        preferred_element_type=jnp.float32)
    res = res + jnp.dot(dih, Wdc[...], preferred_element_type=jnp.float32)
    res = res + jnp.dot(rot(vfx, vfy, vfz, 0), Wix[...],
                        preferred_element_type=jnp.float32)
    res = res + jnp.dot(rot(vfx, vfy, vfz, 1), Wiy[...],
                        preferred_element_type=jnp.float32)
    res = res + jnp.dot(rot(vfx, vfy, vfz, 2), Wiz[...],
                        preferred_element_type=jnp.float32)
    res = res + jnp.dot(aat, Wemb[...], preferred_element_type=jnp.float32)
    res = res + bsum[...]
    out_ref[...] = res[None]


def kernel(traj_pos, traj_backbone_frame, single_mask, single_res_rel,
           aatype, params):
    coords = jnp.nan_to_num(traj_pos[..., :3, :].astype(jnp.float32))
    Bb, Ll = coords.shape[0], coords.shape[1]
    N = Bb * Ll
    padding_mask = single_mask < 0.5
    res_idx = jnp.where(padding_mask, 0, single_res_rel).astype(jnp.float32)
    R = _rotation_frames(coords)
    Rt = jnp.swapaxes(R, -1, -2)
    dih = _dihedral_feats(coords)
    vec = _orientation_vecs(coords)
    confidence = jnp.where(padding_mask, 0.0, 1.0)
    conf_rbf = _rbf(confidence, 0.0, 1.0, 16)
    ca = coords[:, :, 1]

    node_in = jnp.concatenate(
        [dih, conf_rbf, vec[..., 0], vec[..., 1], vec[..., 2], ca,
         res_idx[..., None]], axis=-1).reshape(N, 35)

    ng = params['node_gvp']
    nw = [ng['Wh'], ng['Ws'], ng['bs'][None, :], ng['Wmu'], ng['Wg'],
          ng['bg'][None, :]]
    G = pl.pallas_call(
        _node_body,
        grid=(N // NBN,),
        in_specs=[pl.BlockSpec((NBN, 35), lambda i: (i, 0))] +
                 [pl.BlockSpec(w.shape, functools.partial(
                     lambda r, i: (0,) * r, len(w.shape))) for w in nw],
        out_specs=pl.BlockSpec((NBN, GW), lambda i: (i, 0)),
        out_shape=jax.ShapeDtypeStruct((N, GW), jnp.float32),
        interpret=_INTERPRET,
    )(node_in, *nw)

    misc = jnp.concatenate(
        [dih, vec[..., 0], vec[..., 1], vec[..., 2],
         Rt.reshape(Bb, Ll, 9),
         jax.nn.one_hot(aatype, 22, dtype=jnp.float32) * np.sqrt(EMBED)],
        axis=-1)

    eg = params['edge_gvp']
    lp = params['layers'][0]
    m1, m2, m3 = lp['msg']
    f1, f2 = lp['ff']

    def wpack(g):
        return [g['Wh'], g['Ws'], g['bs'][None, :], g['Wmu'], g['Wg'],
                g['bg'][None, :]]

    Wo = params['W_gvp_out']
    Wi = params['W_gvp_in']
    weights = (wpack(eg) + wpack(m1) + wpack(m2) + wpack(m3) + wpack(f1) +
               wpack(f2) + [
                   Wo[0:NS], Wo[NS + 0::3], Wo[NS + 1::3], Wo[NS + 2::3],
                   params['W_dih'] + Wi[0:6],
                   Wi[6::3], Wi[7::3], Wi[8::3],
                   params['aatype_emb'],
                   (params['b_gvp_out'] + params['b_dih'] +
                    params['b_gvp_in'])[None, :],
               ])

    car = jnp.swapaxes(ca, 1, 2)
    G3 = G.reshape(Bb, Ll, GW)

    wspecs = [pl.BlockSpec(w.shape, functools.partial(
        lambda r, b, i: (0,) * r, len(w.shape))) for w in weights]

    out = pl.pallas_call(
        _edge_body,
        grid=(Bb, Ll // NB),
        in_specs=[
            pl.BlockSpec((1, 3, Ll), lambda b, i: (b, 0, 0)),
            pl.BlockSpec((1, Ll, GW), lambda b, i: (b, 0, 0)),
            pl.BlockSpec((1, NB, GW), lambda b, i: (b, i, 0)),
            pl.BlockSpec((1, NB, 46), lambda b, i: (b, i, 0)),
        ] + wspecs,
        out_specs=pl.BlockSpec((1, NB, EMBED), lambda b, i: (b, i, 0)),
        out_shape=jax.ShapeDtypeStruct((Bb, Ll, EMBED), jnp.float32),
        interpret=_INTERPRET,
    )(car, G3, G3, misc, *weights)
    return out
